# baseline (speedup 1.0000x reference)
import functools

import jax
import jax.numpy as jnp
from jax import lax
from jax.experimental import pallas as pl
from jax.experimental.pallas import tpu as pltpu
from jax.experimental.pallas import tpu_sc as plsc

NC = 2
NS = 16
NW = NC * NS
LANES = 128
GBUF = 2


@functools.partial(jax.jit, static_argnums=(2, 3))
def _emb_lookup(w128, idx_t, n_s, d):
    mesh = plsc.VectorSubcoreMesh(core_axis_name="c", subcore_axis_name="s")
    n_g = d // 8

    @functools.partial(
        pl.kernel,
        mesh=mesh,
        out_type=jax.ShapeDtypeStruct((n_s * n_g * NW, 8, LANES), jnp.float32),
        compiler_params=pltpu.CompilerParams(
            use_tc_tiling_on_sc=False, needs_layout_passes=False
        ),
        scratch_types=[
            pltpu.VMEM((n_s, LANES), jnp.int32),
            pltpu.VMEM((GBUF, LANES, LANES), jnp.float32),
            pltpu.VMEM((GBUF, n_g, 8, LANES), jnp.float32),
            pltpu.VMEM((LANES * (LANES + 1),), jnp.float32),
            pltpu.SemaphoreType.DMA((GBUF,)),
            pltpu.SemaphoreType.DMA((GBUF,)),
        ],
    )
    def body(table_hbm, idx_hbm, p_hbm, idx_v, g_v, p_v, g2_v, gsem, osem):
        wid = lax.axis_index("s") * NC + lax.axis_index("c")
        pltpu.sync_copy(idx_hbm.at[:, pl.ds(wid * LANES, LANES)], idx_v)

        def gather_descr(s, buf):
            return pltpu.make_async_copy(
                table_hbm.at[idx_v.at[s]], g_v.at[buf], gsem.at[buf]
            )

        def out_descr(s, g, buf):
            t = (s * 8 + g) * NW + wid
            return pltpu.make_async_copy(
                p_v.at[buf, g], p_hbm.at[t], osem.at[buf]
            )

        rows_vecs = [
            lax.iota(jnp.int32, 16) + (16 * m) for m in range(LANES // 16)
        ]

        def transpose_task(buf):
            stride = LANES + 1
            iota = lax.iota(jnp.int32, 16)
            iota_stride = iota * stride
            for l in range(LANES):
                for k in range(LANES // 16):
                    vals = g_v[buf, l, pl.ds(16 * k, 16)]
                    addrs = iota + (l * stride + 16 * k)
                    plsc.store_scatter(g2_v, [addrs], vals)
            depth = 6
            seq = [(f, m) for f in range(8 * n_g) for m in range(LANES // 16)]
            pending = {}

            def flush(i):
                v, f2, m2 = pending.pop(i)
                p_v[buf, f2 // 8, f2 % 8, pl.ds(16 * m2, 16)] = v

            for i, (f, m) in enumerate(seq):
                addrs = iota_stride + (16 * m * stride + f)
                pending[i] = (plsc.load_gather(g2_v, [addrs]), f, m)
                if i >= depth:
                    flush(i - depth)
            for i in sorted(pending):
                flush(i)

        for b in range(GBUF):
            gather_descr(b, b).start()

        @pl.loop(0, n_s, step=GBUF)
        def _(s4):
            for k in range(GBUF):
                s = s4 + k
                gather_descr(s, k).wait()

                @pl.when(s >= GBUF)
                def _():
                    for g in range(n_g):
                        out_descr(s - GBUF, g, k).wait()

                transpose_task(k)
                for g in range(n_g):
                    out_descr(s, g, k).start()

                @pl.when(s + GBUF < n_s)
                def _():
                    gather_descr(s + GBUF, k).start()

        for k in range(GBUF):
            for g in range(n_g):
                out_descr(n_s - GBUF + k, g, k).wait()

    return body(w128, idx_t)


def kernel(input, weight):
    s0, s1 = input.shape
    v, d = weight.shape
    w128 = jnp.concatenate(
        [weight, jnp.zeros((v, LANES - d), jnp.float32)], axis=1
    )
    idx_t = input.T.astype(jnp.int32)
    p = _emb_lookup(w128, idx_t, s1, d)
    out = (
        p.reshape(s1, d // 8, s0 // LANES, 8, LANES)
        .transpose(2, 4, 0, 1, 3)
        .reshape(s0, s1, d)
    )
    return out


# speedup vs baseline: 1.7396x; 1.7396x over previous
import functools

import jax
import jax.numpy as jnp
from jax import lax
from jax.experimental import pallas as pl
from jax.experimental.pallas import tpu as pltpu
from jax.experimental.pallas import tpu_sc as plsc

NC = 2
NS = 16
NW = NC * NS
CHUNK = 128
NBUF = 8
HALF = NBUF // 2


@functools.partial(jax.jit, static_argnums=(2, 3))
def _emb_lookup(weight, idx, n_chunks, d):
    mesh = plsc.VectorSubcoreMesh(core_axis_name="c", subcore_axis_name="s")
    b_total = NW * n_chunks * CHUNK

    @functools.partial(
        pl.kernel,
        mesh=mesh,
        out_type=jax.ShapeDtypeStruct((b_total, d), jnp.float32),
        compiler_params=pltpu.CompilerParams(use_tc_tiling_on_sc=False),
        scratch_types=[
            pltpu.VMEM((n_chunks, CHUNK), jnp.int32),
            pltpu.VMEM((NBUF, CHUNK, d), jnp.float32),
            pltpu.SemaphoreType.DMA((NBUF,)),
            pltpu.SemaphoreType.DMA((NBUF,)),
        ],
    )
    def body(table_hbm, idx_hbm, out_hbm, idx_v, rows_v, gsem, osem):
        wid = lax.axis_index("s") * NC + lax.axis_index("c")
        base = wid * (n_chunks * CHUNK)
        pltpu.sync_copy(idx_hbm.at[wid], idx_v)

        def gather_descr(chunk, buf):
            return pltpu.make_async_copy(
                table_hbm.at[idx_v.at[chunk]], rows_v.at[buf], gsem.at[buf]
            )

        def out_descr(chunk, buf):
            return pltpu.make_async_copy(
                rows_v.at[buf],
                out_hbm.at[pl.ds(base + chunk * CHUNK, CHUNK)],
                osem.at[buf],
            )

        for b in range(HALF):
            gather_descr(b, b).start()

        @pl.loop(0, n_chunks, step=NBUF)
        def _(g):
            for b in range(NBUF):
                j = g + b
                gather_descr(j, b).wait()
                out_descr(j, b).start()
                fs = (b + HALF) % NBUF
                fj = j + HALF

                @pl.when(fj < n_chunks)
                def _():
                    @pl.when(fj >= NBUF)
                    def _():
                        out_descr(fj - NBUF, fs).wait()

                    gather_descr(fj, fs).start()

        for b in range(NBUF):
            out_descr(n_chunks - NBUF + b, b).wait()

    return body(weight, idx)


def kernel(input, weight):
    s0, s1 = input.shape
    v, d = weight.shape
    b_total = s0 * s1
    n_chunks = b_total // (NW * CHUNK)
    idx = input.reshape(NW, n_chunks, CHUNK).astype(jnp.int32)
    out = _emb_lookup(weight, idx, n_chunks, d)
    return out.reshape(s0, s1, d)
